# baseline (speedup 1.0000x reference)
import functools

import jax
import jax.numpy as jnp
from jax import lax
from jax.experimental import pallas as pl
from jax.experimental.pallas import tpu as pltpu
from jax.experimental.pallas import tpu_sc as plsc

N = 10000
E = 320000
CIN = 128
COUT = 128
KTOT = 27
NSLOT = 28
NSAMP = 2500
RADIUS = 0.1
MAXN = 64

NW = 32
EPW = E // NW
CHUNK = 400
NB = CHUNK // 16


def _mm_body(x_ref, w_ref, o_ref):
    o_ref[...] = jnp.dot(x_ref[...], w_ref[...],
                         preferred_element_type=jnp.float32)


def _matmul(x, wcat):
    return pl.pallas_call(
        _mm_body,
        grid=(10,),
        in_specs=[pl.BlockSpec((1000, CIN), lambda i: (i, 0)),
                  pl.BlockSpec((CIN, NSLOT * COUT), lambda i: (0, 0))],
        out_specs=pl.BlockSpec((1000, NSLOT * COUT), lambda i: (i, 0)),
        out_shape=jax.ShapeDtypeStruct((N, NSLOT * COUT), jnp.float32),
    )(x, wcat)


def _edge_body(y2, srcr, dstr, a0r, a1r, a2r, outr,
               src_v, dst_v, dstrow_v, a0_v, a1_v, a2_v,
               gidx0_v, gidx1_v, rows0_v, rows1_v, msg_v, zero_v, acc,
               sem0, sem1):
    c = lax.axis_index("c")
    s = lax.axis_index("s")
    wid = c * 16 + s

    start = s * 624
    nch = jnp.where(s == 15, 40, 39)

    z16 = jnp.zeros((16,), jnp.float32)

    def _zrow(r, carry):
        for hh in range(8):
            zero_v[r, pl.ds(hh * 16, 16)] = z16
        return carry

    lax.fori_loop(0, 16, _zrow, 0)

    def _zcopy(i, carry):
        pltpu.sync_copy(zero_v, acc.at[pl.ds(start + i * 16, 16)])
        return carry

    lax.fori_loop(0, nch, _zcopy, 0)
    plsc.subcore_barrier()

    e0 = wid * EPW

    def _fracs(b):
        frs = []
        los = []
        for av in (a0_v, a1_v, a2_v):
            u = av[pl.ds(b * 16, 16)] * jnp.float32(2.0)
            lof = jnp.where(u >= 1.0, jnp.float32(1.0), jnp.float32(0.0))
            los.append(lof.astype(jnp.int32))
            frs.append(jnp.clip(u - lof, 0.0, 1.0))
        return los, frs

    def _prep4(b, g, gidx_ref):
        sv = src_v[pl.ds(b * 16, 16)]
        los, _ = _fracs(b)
        g0 = sv * NSLOT + (los[0] + los[1] * 3 + los[2] * 9)
        for i, cc in enumerate(range(4 * g, 4 * g + 4)):
            b0, b1, b2 = cc & 1, (cc >> 1) & 1, (cc >> 2) & 1
            gidx_ref[pl.ds(i * 16, 16)] = g0 + (b0 + 3 * b1 + 9 * b2)

    def _comp4(b, g, rows_ref):
        _, frs = _fracs(b)
        one = jnp.float32(1.0)
        for e in range(16):
            fr_s = [frs[0][e], frs[1][e], frs[2][e]]
            om_s = [one - f for f in fr_s]
            accs = [jnp.zeros((16,), jnp.float32) for _ in range(8)]
            for i, cc in enumerate(range(4 * g, 4 * g + 4)):
                b0, b1, b2 = cc & 1, (cc >> 1) & 1, (cc >> 2) & 1
                w = ((fr_s[0] if b0 else om_s[0])
                     * (fr_s[1] if b1 else om_s[1])
                     * (fr_s[2] if b2 else om_s[2]))
                wspl = jnp.broadcast_to(w, (16,))
                for hh in range(8):
                    accs[hh] = accs[hh] + wspl * rows_ref[
                        i * 16 + e, pl.ds(hh * 16, 16)]
            for hh in range(8):
                sl = pl.ds(hh * 16, 16)
                if g == 0:
                    msg_v[e, sl] = accs[hh]
                else:
                    msg_v[e, sl] = msg_v[e, sl] + accs[hh]
        if g == 1:
            dstrow_v[0, :] = dst_v[pl.ds(b * 16, 16)]
            pltpu.sync_copy(msg_v, acc.at[dstrow_v.at[0]], add=True)

    def _chunk(k, carry):
        base = e0 + k * CHUNK
        pltpu.sync_copy(srcr.at[pl.ds(base, CHUNK)], src_v)
        pltpu.sync_copy(dstr.at[pl.ds(base, CHUNK)], dst_v)
        pltpu.sync_copy(a0r.at[pl.ds(base, CHUNK)], a0_v)
        pltpu.sync_copy(a1r.at[pl.ds(base, CHUNK)], a1_v)
        pltpu.sync_copy(a2r.at[pl.ds(base, CHUNK)], a2_v)

        _prep4(0, 0, gidx0_v)
        pltpu.async_copy(y2.at[gidx0_v], rows0_v, sem0)

        def _bat(b, carry2):
            _prep4(b, 1, gidx1_v)
            pltpu.async_copy(y2.at[gidx1_v], rows1_v, sem1)
            pltpu.make_async_copy(y2.at[gidx0_v], rows0_v, sem0).wait()
            _comp4(b, 0, rows0_v)
            nb = jnp.minimum(b + 1, NB - 1)
            _prep4(nb, 0, gidx0_v)
            pltpu.async_copy(y2.at[gidx0_v], rows0_v, sem0)
            pltpu.make_async_copy(y2.at[gidx1_v], rows1_v, sem1).wait()
            _comp4(b, 1, rows1_v)
            return carry2

        lax.fori_loop(0, NB, _bat, 0)
        pltpu.make_async_copy(y2.at[gidx0_v], rows0_v, sem0).wait()
        return carry

    lax.fori_loop(0, EPW // CHUNK, _chunk, 0)
    plsc.subcore_barrier()

    def _ocopy(i, carry):
        pltpu.sync_copy(acc.at[pl.ds(start + i * 16, 16)],
                        outr.at[c, pl.ds(start + i * 16, 16)])
        return carry

    lax.fori_loop(0, nch, _ocopy, 0)


def _edge_call(y2, src, dst, a0, a1, a2):
    mesh = plsc.VectorSubcoreMesh(core_axis_name="c", subcore_axis_name="s")
    kern = functools.partial(
        pl.kernel,
        out_type=jax.ShapeDtypeStruct((2, N, COUT), jnp.float32),
        mesh=mesh,
        scratch_types=[
            pltpu.VMEM((CHUNK,), jnp.int32),
            pltpu.VMEM((CHUNK,), jnp.int32),
            pltpu.VMEM((1, 16), jnp.int32),
            pltpu.VMEM((CHUNK,), jnp.float32),
            pltpu.VMEM((CHUNK,), jnp.float32),
            pltpu.VMEM((CHUNK,), jnp.float32),
            pltpu.VMEM((64,), jnp.int32),
            pltpu.VMEM((64,), jnp.int32),
            pltpu.VMEM((64, COUT), jnp.float32),
            pltpu.VMEM((64, COUT), jnp.float32),
            pltpu.VMEM((16, COUT), jnp.float32),
            pltpu.VMEM((16, COUT), jnp.float32),
            pltpu.VMEM_SHARED((N, COUT), jnp.float32),
            pltpu.SemaphoreType.DMA,
            pltpu.SemaphoreType.DMA,
        ],
    )(_edge_body)
    return kern(y2, src, dst, a0, a1, a2)


def _comb_body(p0, p1, yr, b, o):
    v = p0[...] + p1[...] + yr[...] + b[...]
    o[...] = jnp.where(v > 0, v, jnp.exp(v) - 1.0)


def _combine(p0, p1, yroot, bias2d):
    bs = pl.BlockSpec((1000, COUT), lambda i: (i, 0))
    return pl.pallas_call(
        _comb_body,
        grid=(10,),
        in_specs=[bs, bs, bs, pl.BlockSpec((1, COUT), lambda i: (0, 0))],
        out_specs=bs,
        out_shape=jax.ShapeDtypeStruct((N, COUT), jnp.float32),
    )(p0, p1, yroot, bias2d)


def _fps_body(px_ref, py_ref, pz_ref, o_ref):
    px = px_ref[...]
    py = py_ref[...]
    pz = pz_ref[...]
    lin = (lax.broadcasted_iota(jnp.int32, (80, 128), 0) * 128
           + lax.broadcasted_iota(jnp.int32, (80, 128), 1))
    sx = px[0:1, 0:1]
    sy = py[0:1, 0:1]
    sz = pz[0:1, 0:1]
    d = (px - sx) ** 2 + (py - sy) ** 2 + (pz - sz) ** 2
    d = jnp.where(lin < N, d, jnp.float32(-1.0))
    o_ref[pl.ds(0, 1), :] = jnp.zeros((1, 1), jnp.int32)

    def body(i, d):
        m = jnp.max(d)
        nxt = jnp.min(jnp.where(d == m, lin, jnp.int32(2 ** 30)))
        o_ref[pl.ds(i, 1), :] = jnp.reshape(nxt, (1, 1))
        r = nxt // 128
        cc = jnp.remainder(nxt, 128)
        lane = lax.broadcasted_iota(jnp.int32, (1, 128), 1)
        hitl = lane == cc
        sx = jnp.sum(jnp.where(hitl, px_ref[pl.ds(r, 1), :], 0.0))
        sy = jnp.sum(jnp.where(hitl, py_ref[pl.ds(r, 1), :], 0.0))
        sz = jnp.sum(jnp.where(hitl, pz_ref[pl.ds(r, 1), :], 0.0))
        dn = (px - sx) ** 2 + (py - sy) ** 2 + (pz - sz) ** 2
        return jnp.minimum(d, dn)

    lax.fori_loop(1, NSAMP, body, d)


def _fps(px, py, pz):
    return pl.pallas_call(
        _fps_body,
        out_shape=jax.ShapeDtypeStruct((NSAMP, 1), jnp.int32),
    )(px, py, pz)


BQ = 512

def _rad_body(pq_ref, pt_ref, o_ref):
    q = pq_ref[...]
    pt = pt_ref[...]
    q2 = jnp.sum(q * q, axis=1, keepdims=True)
    p2 = jnp.sum(pt * pt, axis=0, keepdims=True)
    d2 = q2 + p2 - 2.0 * jnp.dot(q, pt, preferred_element_type=jnp.float32)
    lane64 = lax.broadcasted_iota(jnp.int32, (BQ, MAXN), 1)
    rr = jnp.float32(RADIUS * RADIUS)
    cols0 = jnp.full((BQ, MAXN), -1, jnp.int32)

    def body(t, carry):
        d2, cols = carry
        ci = lax.broadcasted_iota(jnp.int32, (BQ, N), 1)
        m = jnp.min(d2, axis=1, keepdims=True)
        j = jnp.min(jnp.where(d2 == m, ci, jnp.int32(2 ** 30)),
                    axis=1, keepdims=True)
        sel = jnp.where(m <= rr, j, jnp.int32(-1))
        cols = jnp.where(lane64 == t, sel, cols)
        d2 = jnp.where(ci == j, jnp.float32(1e30), d2)
        return (d2, cols)

    _, cols = lax.fori_loop(0, MAXN, body, (d2, cols0))
    o_ref[...] = cols


def _radius(pq_pad, pt):
    return pl.pallas_call(
        _rad_body,
        grid=(2560 // BQ,),
        in_specs=[pl.BlockSpec((BQ, 3), lambda i: (i, 0)),
                  pl.BlockSpec((3, N), lambda i: (0, 0))],
        out_specs=pl.BlockSpec((BQ, MAXN), lambda i: (i, 0)),
        out_shape=jax.ShapeDtypeStruct((2560, MAXN), jnp.int32),
    )(pq_pad, pt)


def _gath_body(t_hbm, idx_hbm, out_hbm, idx_v, rows_v):
    wid = lax.axis_index("c") * 16 + lax.axis_index("s")
    base = wid * 80
    pltpu.sync_copy(idx_hbm.at[pl.ds(base, 80)], idx_v)
    pltpu.sync_copy(t_hbm.at[idx_v], rows_v)
    pltpu.sync_copy(rows_v, out_hbm.at[pl.ds(base, 80)])


def _gather_rows(table, idxp):
    width = table.shape[1]
    mesh = plsc.VectorSubcoreMesh(core_axis_name="c", subcore_axis_name="s")
    kern = functools.partial(
        pl.kernel,
        out_type=jax.ShapeDtypeStruct((2560, width), jnp.float32),
        mesh=mesh,
        scratch_types=[
            pltpu.VMEM((80,), jnp.int32),
            pltpu.VMEM((80, width), jnp.float32),
        ],
    )(_gath_body)
    return kern(table, idxp)


def kernel(x, edge_index, edge_attr, pos, batch, W, root, bias):
    wcat = jnp.concatenate(
        [jnp.transpose(W, (1, 0, 2)).reshape(CIN, KTOT * COUT), root], axis=1)
    y = _matmul(x, wcat)
    y2 = y.reshape(N * NSLOT, COUT)

    src = edge_index[0]
    dst = edge_index[1]
    parts = _edge_call(y2, src, dst,
                       edge_attr[:, 0], edge_attr[:, 1], edge_attr[:, 2])

    yroot = y[:, KTOT * COUT:]
    h = _combine(parts[0], parts[1], yroot, bias.reshape(1, COUT))

    pp = jnp.pad(pos, ((0, 240), (0, 0)), constant_values=1e6)
    px = pp[:, 0].reshape(80, 128)
    py = pp[:, 1].reshape(80, 128)
    pz = pp[:, 2].reshape(80, 128)
    idx = _fps(px, py, pz)[:, 0]

    idxp = jnp.pad(idx, (0, 2560 - NSAMP))
    batch_f = lax.bitcast_convert_type(batch, jnp.float32).reshape(N, 1)
    table2 = jnp.concatenate(
        [pos, batch_f, edge_attr[:N], jnp.zeros((N, 121), jnp.float32)],
        axis=1)
    g = _gather_rows(table2, idxp)
    pos_new = g[:NSAMP, :3]
    batch_new = lax.bitcast_convert_type(g[:NSAMP, 3], jnp.int32)
    ea_new = g[:NSAMP, 4:7]

    x_new = _gather_rows(h, idxp)[:NSAMP]

    col_pad = _radius(g[:, :3], jnp.transpose(pos))
    col = col_pad[:NSAMP]
    row = jnp.broadcast_to(jnp.arange(NSAMP, dtype=jnp.int32)[:, None],
                           (NSAMP, MAXN))
    edge_index_new = jnp.stack([col.reshape(-1), row.reshape(-1)], axis=0)
    return (x_new, edge_index_new, pos_new, batch_new, ea_new)


# speedup vs baseline: 1.0100x; 1.0100x over previous
import functools

import jax
import jax.numpy as jnp
from jax import lax
from jax.experimental import pallas as pl
from jax.experimental.pallas import tpu as pltpu
from jax.experimental.pallas import tpu_sc as plsc

N = 10000
E = 320000
CIN = 128
COUT = 128
KTOT = 27
NSLOT = 28
NSAMP = 2500
RADIUS = 0.1
MAXN = 64

NW = 32
EPW = E // NW
CHUNK = 400
NB = CHUNK // 16


def _mm_body(x_ref, w_ref, o_ref):
    o_ref[...] = jnp.dot(x_ref[...], w_ref[...],
                         preferred_element_type=jnp.float32)


def _matmul(x, wcat):
    return pl.pallas_call(
        _mm_body,
        grid=(10,),
        in_specs=[pl.BlockSpec((1000, CIN), lambda i: (i, 0)),
                  pl.BlockSpec((CIN, NSLOT * COUT), lambda i: (0, 0))],
        out_specs=pl.BlockSpec((1000, NSLOT * COUT), lambda i: (i, 0)),
        out_shape=jax.ShapeDtypeStruct((N, NSLOT * COUT), jnp.float32),
    )(x, wcat)


def _edge_body(y2, srcr, dstr, a0r, a1r, a2r, outr,
               src_v, dst_v, dstrow_v, a0_v, a1_v, a2_v,
               gidx0_v, gidx1_v, rows0_v, rows1_v, msg_v, zero_v, acc,
               sem0, sem1):
    c = lax.axis_index("c")
    s = lax.axis_index("s")
    wid = c * 16 + s

    start = s * 624
    nch = jnp.where(s == 15, 40, 39)

    z16 = jnp.zeros((16,), jnp.float32)

    def _zrow(r, carry):
        for hh in range(8):
            zero_v[r, pl.ds(hh * 16, 16)] = z16
        return carry

    lax.fori_loop(0, 16, _zrow, 0)

    def _zcopy(i, carry):
        pltpu.sync_copy(zero_v, acc.at[pl.ds(start + i * 16, 16)])
        return carry

    lax.fori_loop(0, nch, _zcopy, 0)
    plsc.subcore_barrier()

    e0 = wid * EPW

    def _fracs(b):
        frs = []
        los = []
        for av in (a0_v, a1_v, a2_v):
            u = av[pl.ds(b * 16, 16)] * jnp.float32(2.0)
            lof = jnp.where(u >= 1.0, jnp.float32(1.0), jnp.float32(0.0))
            los.append(lof.astype(jnp.int32))
            frs.append(jnp.clip(u - lof, 0.0, 1.0))
        return los, frs

    def _prep4(b, g, gidx_ref):
        sv = src_v[pl.ds(b * 16, 16)]
        los, _ = _fracs(b)
        g0 = sv * NSLOT + (los[0] + los[1] * 3 + los[2] * 9)
        for i, cc in enumerate(range(4 * g, 4 * g + 4)):
            b0, b1, b2 = cc & 1, (cc >> 1) & 1, (cc >> 2) & 1
            gidx_ref[pl.ds(i * 16, 16)] = g0 + (b0 + 3 * b1 + 9 * b2)

    def _comp4(b, g, rows_ref):
        _, frs = _fracs(b)
        one = jnp.float32(1.0)
        for e in range(16):
            fr_s = [frs[0][e], frs[1][e], frs[2][e]]
            om_s = [one - f for f in fr_s]
            accs = [jnp.zeros((16,), jnp.float32) for _ in range(8)]
            for i, cc in enumerate(range(4 * g, 4 * g + 4)):
                b0, b1, b2 = cc & 1, (cc >> 1) & 1, (cc >> 2) & 1
                w = ((fr_s[0] if b0 else om_s[0])
                     * (fr_s[1] if b1 else om_s[1])
                     * (fr_s[2] if b2 else om_s[2]))
                wspl = jnp.broadcast_to(w, (16,))
                for hh in range(8):
                    accs[hh] = accs[hh] + wspl * rows_ref[
                        i * 16 + e, pl.ds(hh * 16, 16)]
            for hh in range(8):
                sl = pl.ds(hh * 16, 16)
                if g == 0:
                    msg_v[e, sl] = accs[hh]
                else:
                    msg_v[e, sl] = msg_v[e, sl] + accs[hh]
        if g == 1:
            dstrow_v[0, :] = dst_v[pl.ds(b * 16, 16)]
            pltpu.sync_copy(msg_v, acc.at[dstrow_v.at[0]], add=True)

    def _chunk(k, carry):
        base = e0 + k * CHUNK
        pltpu.sync_copy(srcr.at[pl.ds(base, CHUNK)], src_v)
        pltpu.sync_copy(dstr.at[pl.ds(base, CHUNK)], dst_v)
        pltpu.sync_copy(a0r.at[pl.ds(base, CHUNK)], a0_v)
        pltpu.sync_copy(a1r.at[pl.ds(base, CHUNK)], a1_v)
        pltpu.sync_copy(a2r.at[pl.ds(base, CHUNK)], a2_v)

        _prep4(0, 0, gidx0_v)
        pltpu.async_copy(y2.at[gidx0_v], rows0_v, sem0)

        def _bat(b, carry2):
            _prep4(b, 1, gidx1_v)
            pltpu.async_copy(y2.at[gidx1_v], rows1_v, sem1)
            pltpu.make_async_copy(y2.at[gidx0_v], rows0_v, sem0).wait()
            _comp4(b, 0, rows0_v)
            nb = jnp.minimum(b + 1, NB - 1)
            _prep4(nb, 0, gidx0_v)
            pltpu.async_copy(y2.at[gidx0_v], rows0_v, sem0)
            pltpu.make_async_copy(y2.at[gidx1_v], rows1_v, sem1).wait()
            _comp4(b, 1, rows1_v)
            return carry2

        lax.fori_loop(0, NB, _bat, 0)
        pltpu.make_async_copy(y2.at[gidx0_v], rows0_v, sem0).wait()
        return carry

    lax.fori_loop(0, EPW // CHUNK, _chunk, 0)
    plsc.subcore_barrier()

    def _ocopy(i, carry):
        pltpu.sync_copy(acc.at[pl.ds(start + i * 16, 16)],
                        outr.at[c, pl.ds(start + i * 16, 16)])
        return carry

    lax.fori_loop(0, nch, _ocopy, 0)


def _edge_call(y2, src, dst, a0, a1, a2):
    mesh = plsc.VectorSubcoreMesh(core_axis_name="c", subcore_axis_name="s")
    kern = functools.partial(
        pl.kernel,
        out_type=jax.ShapeDtypeStruct((2, N, COUT), jnp.float32),
        mesh=mesh,
        scratch_types=[
            pltpu.VMEM((CHUNK,), jnp.int32),
            pltpu.VMEM((CHUNK,), jnp.int32),
            pltpu.VMEM((1, 16), jnp.int32),
            pltpu.VMEM((CHUNK,), jnp.float32),
            pltpu.VMEM((CHUNK,), jnp.float32),
            pltpu.VMEM((CHUNK,), jnp.float32),
            pltpu.VMEM((64,), jnp.int32),
            pltpu.VMEM((64,), jnp.int32),
            pltpu.VMEM((64, COUT), jnp.float32),
            pltpu.VMEM((64, COUT), jnp.float32),
            pltpu.VMEM((16, COUT), jnp.float32),
            pltpu.VMEM((16, COUT), jnp.float32),
            pltpu.VMEM_SHARED((N, COUT), jnp.float32),
            pltpu.SemaphoreType.DMA,
            pltpu.SemaphoreType.DMA,
        ],
    )(_edge_body)
    return kern(y2, src, dst, a0, a1, a2)


def _comb_body(p0, p1, yr, b, o):
    v = p0[...] + p1[...] + yr[...] + b[...]
    o[...] = jnp.where(v > 0, v, jnp.exp(v) - 1.0)


def _combine(p0, p1, yroot, bias2d):
    bs = pl.BlockSpec((1000, COUT), lambda i: (i, 0))
    return pl.pallas_call(
        _comb_body,
        grid=(10,),
        in_specs=[bs, bs, bs, pl.BlockSpec((1, COUT), lambda i: (0, 0))],
        out_specs=bs,
        out_shape=jax.ShapeDtypeStruct((N, COUT), jnp.float32),
    )(p0, p1, yroot, bias2d)


def _fps_body(px_ref, py_ref, pz_ref, o_ref):
    px = px_ref[...]
    py = py_ref[...]
    pz = pz_ref[...]
    lin = (lax.broadcasted_iota(jnp.int32, (80, 128), 0) * 128
           + lax.broadcasted_iota(jnp.int32, (80, 128), 1))
    sx = px[0:1, 0:1]
    sy = py[0:1, 0:1]
    sz = pz[0:1, 0:1]
    d = (px - sx) ** 2 + (py - sy) ** 2 + (pz - sz) ** 2
    d = jnp.where(lin < N, d, jnp.float32(-1.0))
    o_ref[pl.ds(0, 1), :] = jnp.zeros((1, 1), jnp.int32)

    def body(i, d):
        m = jnp.max(d)
        nxt = jnp.min(jnp.where(d == m, lin, jnp.int32(2 ** 30)))
        o_ref[pl.ds(i, 1), :] = jnp.reshape(nxt, (1, 1))
        r = nxt // 128
        cc = jnp.remainder(nxt, 128)
        lane = lax.broadcasted_iota(jnp.int32, (1, 128), 1)
        hitl = lane == cc
        sx = jnp.sum(jnp.where(hitl, px_ref[pl.ds(r, 1), :], 0.0))
        sy = jnp.sum(jnp.where(hitl, py_ref[pl.ds(r, 1), :], 0.0))
        sz = jnp.sum(jnp.where(hitl, pz_ref[pl.ds(r, 1), :], 0.0))
        dn = (px - sx) ** 2 + (py - sy) ** 2 + (pz - sz) ** 2
        return jnp.minimum(d, dn)

    lax.fori_loop(1, NSAMP, body, d)


def _fps(px, py, pz):
    return pl.pallas_call(
        _fps_body,
        out_shape=jax.ShapeDtypeStruct((NSAMP, 1), jnp.int32),
    )(px, py, pz)


BQ = 256

def _rad_body(pq_ref, pt_ref, o_ref):
    q = pq_ref[...]
    pt = pt_ref[...]
    q2 = jnp.sum(q * q, axis=1, keepdims=True)
    p2 = jnp.sum(pt * pt, axis=0, keepdims=True)
    d2 = q2 + p2 - 2.0 * jnp.dot(q, pt, preferred_element_type=jnp.float32)
    lane64 = lax.broadcasted_iota(jnp.int32, (BQ, MAXN), 1)
    rr = jnp.float32(RADIUS * RADIUS)
    cols0 = jnp.full((BQ, MAXN), -1, jnp.int32)

    def body(t, carry):
        d2, cols = carry
        ci = lax.broadcasted_iota(jnp.int32, (BQ, N), 1)
        m = jnp.min(d2, axis=1, keepdims=True)
        j = jnp.min(jnp.where(d2 == m, ci, jnp.int32(2 ** 30)),
                    axis=1, keepdims=True)
        sel = jnp.where(m <= rr, j, jnp.int32(-1))
        cols = jnp.where(lane64 == t, sel, cols)
        d2 = jnp.where(ci == j, jnp.float32(1e30), d2)
        return (d2, cols)

    _, cols = lax.fori_loop(0, MAXN, body, (d2, cols0))
    o_ref[...] = cols


def _radius(pq_pad, pt):
    return pl.pallas_call(
        _rad_body,
        grid=(2560 // BQ,),
        in_specs=[pl.BlockSpec((BQ, 3), lambda i: (i, 0)),
                  pl.BlockSpec((3, N), lambda i: (0, 0))],
        out_specs=pl.BlockSpec((BQ, MAXN), lambda i: (i, 0)),
        out_shape=jax.ShapeDtypeStruct((2560, MAXN), jnp.int32),
    )(pq_pad, pt)


def _gath_body(t_hbm, idx_hbm, out_hbm, idx_v, rows_v):
    wid = lax.axis_index("c") * 16 + lax.axis_index("s")
    base = wid * 80
    pltpu.sync_copy(idx_hbm.at[pl.ds(base, 80)], idx_v)
    pltpu.sync_copy(t_hbm.at[idx_v], rows_v)
    pltpu.sync_copy(rows_v, out_hbm.at[pl.ds(base, 80)])


def _gather_rows(table, idxp):
    width = table.shape[1]
    mesh = plsc.VectorSubcoreMesh(core_axis_name="c", subcore_axis_name="s")
    kern = functools.partial(
        pl.kernel,
        out_type=jax.ShapeDtypeStruct((2560, width), jnp.float32),
        mesh=mesh,
        scratch_types=[
            pltpu.VMEM((80,), jnp.int32),
            pltpu.VMEM((80, width), jnp.float32),
        ],
    )(_gath_body)
    return kern(table, idxp)


def kernel(x, edge_index, edge_attr, pos, batch, W, root, bias):
    wcat = jnp.concatenate(
        [jnp.transpose(W, (1, 0, 2)).reshape(CIN, KTOT * COUT), root], axis=1)
    y = _matmul(x, wcat)
    y2 = y.reshape(N * NSLOT, COUT)

    src = edge_index[0]
    dst = edge_index[1]
    parts = _edge_call(y2, src, dst,
                       edge_attr[:, 0], edge_attr[:, 1], edge_attr[:, 2])

    yroot = y[:, KTOT * COUT:]
    h = _combine(parts[0], parts[1], yroot, bias.reshape(1, COUT))

    pp = jnp.pad(pos, ((0, 240), (0, 0)), constant_values=1e6)
    px = pp[:, 0].reshape(80, 128)
    py = pp[:, 1].reshape(80, 128)
    pz = pp[:, 2].reshape(80, 128)
    idx = _fps(px, py, pz)[:, 0]

    idxp = jnp.pad(idx, (0, 2560 - NSAMP))
    batch_f = lax.bitcast_convert_type(batch, jnp.float32).reshape(N, 1)
    table2 = jnp.concatenate(
        [pos, batch_f, edge_attr[:N], jnp.zeros((N, 121), jnp.float32)],
        axis=1)
    g = _gather_rows(table2, idxp)
    pos_new = g[:NSAMP, :3]
    batch_new = lax.bitcast_convert_type(g[:NSAMP, 3], jnp.int32)
    ea_new = g[:NSAMP, 4:7]

    x_new = _gather_rows(h, idxp)[:NSAMP]

    col_pad = _radius(g[:, :3], jnp.transpose(pos))
    col = col_pad[:NSAMP]
    row = jnp.broadcast_to(jnp.arange(NSAMP, dtype=jnp.int32)[:, None],
                           (NSAMP, MAXN))
    edge_index_new = jnp.stack([col.reshape(-1), row.reshape(-1)], axis=0)
    return (x_new, edge_index_new, pos_new, batch_new, ea_new)
